# Initial kernel scaffold; baseline (speedup 1.0000x reference)
import jax
import jax.numpy as jnp
from jax.experimental import pallas as pl


def kernel(x, label, w1, b1, w2, b2, w3, b3, w4, b4):
    raise NotImplementedError("write your pallas kernel here")


# speedup vs baseline: 2.9543x; 2.9543x over previous
import functools

import jax
import jax.numpy as jnp
from jax import lax
from jax.experimental import pallas as pl
from jax.experimental.pallas import tpu as pltpu

_H = 384
_W = 384
_WP = 392
_TH = 8
_THRESHOLD = 100.0


def _conv_body(prev_ref, cur_ref, next_ref, wm_ref, b_ref, out_ref, slab_ref,
               *, cin, cout):
    i = pl.program_id(1)
    nh = pl.num_programs(1)

    slab_ref[1:_TH + 1] = cur_ref[0]

    @pl.when(i == 0)
    def _():
        slab_ref[0:1] = jnp.zeros((1, _WP, cin), jnp.float32)

    @pl.when(i > 0)
    def _():
        slab_ref[0:1] = prev_ref[0, _TH - 1:_TH]

    @pl.when(i == nh - 1)
    def _():
        slab_ref[_TH + 1:_TH + 2] = jnp.zeros((1, _WP, cin), jnp.float32)

    @pl.when(i < nh - 1)
    def _():
        slab_ref[_TH + 1:_TH + 2] = next_ref[0, 0:1]

    cat = jnp.concatenate(
        [slab_ref[0:_TH], slab_ref[1:_TH + 1], slab_ref[2:_TH + 2]], axis=-1)
    y = jnp.dot(cat.reshape(_TH * _WP, 3 * cin).astype(jnp.bfloat16),
                wm_ref[...],
                preferred_element_type=jnp.float32).reshape(_TH, _WP, 3 * cout)
    y0 = y[:, :, 0:cout]
    y1 = y[:, :, cout:2 * cout]
    y2 = y[:, :, 2 * cout:3 * cout]
    s = (jnp.roll(y0, 1, axis=1) + y1 + jnp.roll(y2, -1, axis=1)
         + b_ref[0][None, None, :])
    act = jnp.maximum(s, 0.0)
    col = lax.broadcasted_iota(jnp.int32, (_TH, _WP, cout), 1)
    out_ref[0] = jnp.where((col >= 1) & (col <= _W), act, 0.0)


def _conv_layer(xpad, wm, bias, cin, cout):
    b = xpad.shape[0]
    nh = _H // _TH
    spec_in = lambda f: pl.BlockSpec((1, _TH, _WP, cin), f)
    return pl.pallas_call(
        functools.partial(_conv_body, cin=cin, cout=cout),
        grid=(b, nh),
        in_specs=[
            spec_in(lambda bb, i: (bb, jnp.maximum(i - 1, 0), 0, 0)),
            spec_in(lambda bb, i: (bb, i, 0, 0)),
            spec_in(lambda bb, i: (bb, jnp.minimum(i + 1, nh - 1), 0, 0)),
            pl.BlockSpec((3 * cin, 3 * cout), lambda bb, i: (0, 0)),
            pl.BlockSpec((1, cout), lambda bb, i: (0, 0)),
        ],
        out_specs=pl.BlockSpec((1, _TH, _WP, cout), lambda bb, i: (bb, i, 0, 0)),
        out_shape=jax.ShapeDtypeStruct((b, _H, _WP, cout), jnp.float32),
        scratch_shapes=[pltpu.VMEM((_TH + 2, _WP, cin), jnp.float32)],
        compiler_params=pltpu.CompilerParams(
            dimension_semantics=("parallel", "parallel")),
    )(xpad, xpad, xpad, wm, bias)


def _head_body(prev_ref, cur_ref, next_ref, wm_ref, b_ref, w4x_ref, w4y_ref,
               b4_ref, dx_ref, dy_ref, slab_ref, *, cin, cout):
    i = pl.program_id(1)
    nh = pl.num_programs(1)

    slab_ref[1:_TH + 1] = cur_ref[0]

    @pl.when(i == 0)
    def _():
        slab_ref[0:1] = jnp.zeros((1, _WP, cin), jnp.float32)

    @pl.when(i > 0)
    def _():
        slab_ref[0:1] = prev_ref[0, _TH - 1:_TH]

    @pl.when(i == nh - 1)
    def _():
        slab_ref[_TH + 1:_TH + 2] = jnp.zeros((1, _WP, cin), jnp.float32)

    @pl.when(i < nh - 1)
    def _():
        slab_ref[_TH + 1:_TH + 2] = next_ref[0, 0:1]

    cat = jnp.concatenate(
        [slab_ref[0:_TH], slab_ref[1:_TH + 1], slab_ref[2:_TH + 2]], axis=-1)
    y = jnp.dot(cat.reshape(_TH * _WP, 3 * cin).astype(jnp.bfloat16),
                wm_ref[...],
                preferred_element_type=jnp.float32).reshape(_TH, _WP, 3 * cout)
    y0 = y[:, :, 0:cout]
    y1 = y[:, :, cout:2 * cout]
    y2 = y[:, :, 2 * cout:3 * cout]
    s = (jnp.roll(y0, 1, axis=1) + y1 + jnp.roll(y2, -1, axis=1)
         + b_ref[0][None, None, :])
    h3 = jnp.maximum(s, 0.0)
    h3b = h3.astype(jnp.bfloat16).astype(jnp.float32)
    d0 = jnp.sum(h3b * w4x_ref[0][None, None, :], axis=-1) + b4_ref[0]
    d1 = jnp.sum(h3b * w4y_ref[0][None, None, :], axis=-1) + b4_ref[1]
    d0s = d0[:, 1:_W + 1]
    d1s = d1[:, 1:_W + 1]
    norm = jnp.sqrt(d0s * d0s + d1s * d1s)
    denom = jnp.maximum(norm, 1e-12)
    dx_ref[0] = d0s / denom
    dy_ref[0] = d1s / denom


def _head_layer(xpad, wm, bias, w4x, w4y, b4, cin, cout):
    b = xpad.shape[0]
    nh = _H // _TH
    spec_in = lambda f: pl.BlockSpec((1, _TH, _WP, cin), f)
    out = jax.ShapeDtypeStruct((b, _H, _W), jnp.float32)
    return pl.pallas_call(
        functools.partial(_head_body, cin=cin, cout=cout),
        grid=(b, nh),
        in_specs=[
            spec_in(lambda bb, i: (bb, jnp.maximum(i - 1, 0), 0, 0)),
            spec_in(lambda bb, i: (bb, i, 0, 0)),
            spec_in(lambda bb, i: (bb, jnp.minimum(i + 1, nh - 1), 0, 0)),
            pl.BlockSpec((3 * cin, 3 * cout), lambda bb, i: (0, 0)),
            pl.BlockSpec((1, cout), lambda bb, i: (0, 0)),
            pl.BlockSpec((1, cout), lambda bb, i: (0, 0)),
            pl.BlockSpec((1, cout), lambda bb, i: (0, 0)),
            pl.BlockSpec(memory_space=pltpu.SMEM),
        ],
        out_specs=(
            pl.BlockSpec((1, _TH, _W), lambda bb, i: (bb, i, 0)),
            pl.BlockSpec((1, _TH, _W), lambda bb, i: (bb, i, 0)),
        ),
        out_shape=(out, out),
        scratch_shapes=[pltpu.VMEM((_TH + 2, _WP, cin), jnp.float32)],
        compiler_params=pltpu.CompilerParams(
            dimension_semantics=("parallel", "parallel")),
    )(xpad, xpad, xpad, wm, bias, w4x, w4y, b4)


def _roll2(t, j, k):
    if j:
        t = jnp.roll(t, j, axis=0)
    if k:
        t = jnp.roll(t, k, axis=1)
    return t


def _centers_body(label_ref, dx_ref, dy_ref, out_ref):
    lab = label_ref[0]
    dx = dx_ref[0]
    dy = dy_ref[0]
    xi = lax.broadcasted_iota(jnp.int32, (_H, _W), 1)
    yi = lax.broadcasted_iota(jnp.int32, (_H, _W), 0)
    xs = xi.astype(jnp.float32)
    ys = yi.astype(jnp.float32)
    vx = jnp.clip(xs + dx, 0.0, float(_W - 1)).astype(jnp.int32)
    vy = jnp.clip(ys + dy, 0.0, float(_H - 1)).astype(jnp.int32)
    rx = vx - xi
    ry = vy - yi
    flat = yi * _W + xi
    for oi, obj in enumerate((1, 2)):
        hough = jnp.zeros((_H, _W), jnp.float32)
        for j in (-1, 0, 1):
            for k in (-1, 0, 1):
                t = jnp.where((lab == obj) & (ry == j) & (rx == k), 1.0, 0.0)
                hough = hough + _roll2(t, j, k)
        mx = jnp.max(hough)
        idx = jnp.min(jnp.where(hough == mx, flat, _H * _W))
        found = mx > _THRESHOLD
        out_ref[0, oi, 0] = jnp.where(found, idx % _W, -1)
        out_ref[0, oi, 1] = jnp.where(found, idx // _W, -1)


def _centers(label, dxa, dya):
    b = label.shape[0]
    return pl.pallas_call(
        _centers_body,
        grid=(b,),
        in_specs=[
            pl.BlockSpec((1, _H, _W), lambda bb: (bb, 0, 0)),
            pl.BlockSpec((1, _H, _W), lambda bb: (bb, 0, 0)),
            pl.BlockSpec((1, _H, _W), lambda bb: (bb, 0, 0)),
        ],
        out_specs=pl.BlockSpec((1, 2, 2), lambda bb: (bb, 0, 0),
                               memory_space=pltpu.SMEM),
        out_shape=jax.ShapeDtypeStruct((b, 2, 2), jnp.int32),
        compiler_params=pltpu.CompilerParams(
            dimension_semantics=("parallel",)),
    )(label, dxa, dya)


def _wmat(w):
    return jnp.transpose(w, (2, 1, 3, 0)).reshape(3 * w.shape[1], 3 * w.shape[0])


def kernel(x, label, w1, b1, w2, b2, w3, b3, w4, b4):
    xt = jnp.transpose(x, (0, 2, 3, 1))
    xp = jnp.pad(xt, ((0, 0), (0, 0), (1, _WP - _W - 1), (0, 0)))
    h1 = _conv_layer(xp, _wmat(w1).astype(jnp.bfloat16), b1.reshape(1, -1), 96, 64)
    h2 = _conv_layer(h1, _wmat(w2).astype(jnp.bfloat16), b2.reshape(1, -1), 64, 64)
    w4r = w4.reshape(2, 32)
    w4rr = w4r.astype(jnp.bfloat16).astype(jnp.float32)
    dxa, dya = _head_layer(h2, _wmat(w3).astype(jnp.bfloat16), b3.reshape(1, -1),
                           w4rr[0:1], w4rr[1:2], b4, 64, 32)
    direction_vectors = jnp.stack([dxa, dya], axis=1)
    centers = _centers(label, dxa, dya)
    return direction_vectors, centers


# speedup vs baseline: 4.2229x; 1.4294x over previous
import functools

import jax
import jax.numpy as jnp
from jax import lax
from jax.experimental import pallas as pl
from jax.experimental.pallas import tpu as pltpu

_H = 384
_W = 384
_TH = 16
_THRESHOLD = 100.0


def _build_cat(prev_ref, cur_ref, next_ref, slab_ref, cin):
    i = pl.program_id(1)
    nh = pl.num_programs(1)

    slab_ref[1:_TH + 1] = cur_ref[0]

    @pl.when(i == 0)
    def _():
        slab_ref[0:1] = jnp.zeros((1, _W, cin), slab_ref.dtype)

    @pl.when(i > 0)
    def _():
        slab_ref[0:1] = prev_ref[0, _TH - 1:_TH]

    @pl.when(i == nh - 1)
    def _():
        slab_ref[_TH + 1:_TH + 2] = jnp.zeros((1, _W, cin), slab_ref.dtype)

    @pl.when(i < nh - 1)
    def _():
        slab_ref[_TH + 1:_TH + 2] = next_ref[0, 0:1]

    return jnp.concatenate(
        [slab_ref[0:_TH], slab_ref[1:_TH + 1], slab_ref[2:_TH + 2]], axis=-1)


def _conv_matmul(cat, wm_ref, b_ref, cin, cout):
    y = jnp.dot(cat.reshape(_TH * _W, 3 * cin).astype(jnp.bfloat16),
                wm_ref[...],
                preferred_element_type=jnp.float32).reshape(_TH, _W, 3 * cout)
    y0 = y[:, :, 0:cout]
    y1 = y[:, :, cout:2 * cout]
    y2 = y[:, :, 2 * cout:3 * cout]
    col = lax.broadcasted_iota(jnp.int32, (_TH, _W, cout), 1)
    left = jnp.where(col >= 1, jnp.roll(y0, 1, axis=1), 0.0)
    right = jnp.where(col <= _W - 2, jnp.roll(y2, -1, axis=1), 0.0)
    return left + y1 + right + b_ref[0][None, None, :]


def _conv_body(prev_ref, cur_ref, next_ref, wm_ref, b_ref, out_ref, slab_ref,
               *, cin, cout):
    cat = _build_cat(prev_ref, cur_ref, next_ref, slab_ref, cin)
    s = _conv_matmul(cat, wm_ref, b_ref, cin, cout)
    out_ref[0] = jnp.maximum(s, 0.0).astype(jnp.bfloat16)


def _conv_layer(xpad, wm, bias, cin, cout):
    b = xpad.shape[0]
    nh = _H // _TH
    dt = xpad.dtype
    spec_in = lambda f: pl.BlockSpec((1, _TH, _W, cin), f)
    return pl.pallas_call(
        functools.partial(_conv_body, cin=cin, cout=cout),
        grid=(b, nh),
        in_specs=[
            spec_in(lambda bb, i: (bb, jnp.maximum(i - 1, 0), 0, 0)),
            spec_in(lambda bb, i: (bb, i, 0, 0)),
            spec_in(lambda bb, i: (bb, jnp.minimum(i + 1, nh - 1), 0, 0)),
            pl.BlockSpec((3 * cin, 3 * cout), lambda bb, i: (0, 0)),
            pl.BlockSpec((1, cout), lambda bb, i: (0, 0)),
        ],
        out_specs=pl.BlockSpec((1, _TH, _W, cout), lambda bb, i: (bb, i, 0, 0)),
        out_shape=jax.ShapeDtypeStruct((b, _H, _W, cout), jnp.bfloat16),
        scratch_shapes=[pltpu.VMEM((_TH + 2, _W, cin), dt)],
        compiler_params=pltpu.CompilerParams(
            dimension_semantics=("parallel", "parallel")),
    )(xpad, xpad, xpad, wm, bias)


def _head_body(prev_ref, cur_ref, next_ref, wm_ref, b_ref, w4x_ref, w4y_ref,
               b4_ref, dx_ref, dy_ref, slab_ref, *, cin, cout):
    cat = _build_cat(prev_ref, cur_ref, next_ref, slab_ref, cin)
    s = _conv_matmul(cat, wm_ref, b_ref, cin, cout)
    h3 = jnp.maximum(s, 0.0)
    h3b = h3.astype(jnp.bfloat16).astype(jnp.float32)
    d0 = jnp.sum(h3b * w4x_ref[0][None, None, :], axis=-1) + b4_ref[0]
    d1 = jnp.sum(h3b * w4y_ref[0][None, None, :], axis=-1) + b4_ref[1]
    norm = jnp.sqrt(d0 * d0 + d1 * d1)
    denom = jnp.maximum(norm, 1e-12)
    dx_ref[0] = d0 / denom
    dy_ref[0] = d1 / denom


def _head_layer(xpad, wm, bias, w4x, w4y, b4, cin, cout):
    b = xpad.shape[0]
    nh = _H // _TH
    spec_in = lambda f: pl.BlockSpec((1, _TH, _W, cin), f)
    out = jax.ShapeDtypeStruct((b, _H, _W), jnp.float32)
    return pl.pallas_call(
        functools.partial(_head_body, cin=cin, cout=cout),
        grid=(b, nh),
        in_specs=[
            spec_in(lambda bb, i: (bb, jnp.maximum(i - 1, 0), 0, 0)),
            spec_in(lambda bb, i: (bb, i, 0, 0)),
            spec_in(lambda bb, i: (bb, jnp.minimum(i + 1, nh - 1), 0, 0)),
            pl.BlockSpec((3 * cin, 3 * cout), lambda bb, i: (0, 0)),
            pl.BlockSpec((1, cout), lambda bb, i: (0, 0)),
            pl.BlockSpec((1, cout), lambda bb, i: (0, 0)),
            pl.BlockSpec((1, cout), lambda bb, i: (0, 0)),
            pl.BlockSpec(memory_space=pltpu.SMEM),
        ],
        out_specs=(
            pl.BlockSpec((1, _TH, _W), lambda bb, i: (bb, i, 0)),
            pl.BlockSpec((1, _TH, _W), lambda bb, i: (bb, i, 0)),
        ),
        out_shape=(out, out),
        scratch_shapes=[pltpu.VMEM((_TH + 2, _W, cin), jnp.bfloat16)],
        compiler_params=pltpu.CompilerParams(
            dimension_semantics=("parallel", "parallel")),
    )(xpad, xpad, xpad, wm, bias, w4x, w4y, b4)


def _roll2(t, j, k):
    if j:
        t = jnp.roll(t, j, axis=0)
    if k:
        t = jnp.roll(t, k, axis=1)
    return t


def _centers_body(label_ref, dx_ref, dy_ref, out_ref):
    lab = label_ref[0]
    dx = dx_ref[0]
    dy = dy_ref[0]
    xi = lax.broadcasted_iota(jnp.int32, (_H, _W), 1)
    yi = lax.broadcasted_iota(jnp.int32, (_H, _W), 0)
    xs = xi.astype(jnp.float32)
    ys = yi.astype(jnp.float32)
    vx = jnp.clip(xs + dx, 0.0, float(_W - 1)).astype(jnp.int32)
    vy = jnp.clip(ys + dy, 0.0, float(_H - 1)).astype(jnp.int32)
    rx = vx - xi
    ry = vy - yi
    flat = yi * _W + xi
    for oi, obj in enumerate((1, 2)):
        hough = jnp.zeros((_H, _W), jnp.float32)
        for j in (-1, 0, 1):
            for k in (-1, 0, 1):
                t = jnp.where((lab == obj) & (ry == j) & (rx == k), 1.0, 0.0)
                hough = hough + _roll2(t, j, k)
        mx = jnp.max(hough)
        idx = jnp.min(jnp.where(hough == mx, flat, _H * _W))
        found = mx > _THRESHOLD
        out_ref[0, oi, 0] = jnp.where(found, idx % _W, -1)
        out_ref[0, oi, 1] = jnp.where(found, idx // _W, -1)


def _centers(label, dxa, dya):
    b = label.shape[0]
    return pl.pallas_call(
        _centers_body,
        grid=(b,),
        in_specs=[
            pl.BlockSpec((1, _H, _W), lambda bb: (bb, 0, 0)),
            pl.BlockSpec((1, _H, _W), lambda bb: (bb, 0, 0)),
            pl.BlockSpec((1, _H, _W), lambda bb: (bb, 0, 0)),
        ],
        out_specs=pl.BlockSpec((1, 2, 2), lambda bb: (bb, 0, 0),
                               memory_space=pltpu.SMEM),
        out_shape=jax.ShapeDtypeStruct((b, 2, 2), jnp.int32),
        compiler_params=pltpu.CompilerParams(
            dimension_semantics=("parallel",)),
    )(label, dxa, dya)


def _wmat(w):
    return jnp.transpose(w, (2, 1, 3, 0)).reshape(
        3 * w.shape[1], 3 * w.shape[0]).astype(jnp.bfloat16)


def kernel(x, label, w1, b1, w2, b2, w3, b3, w4, b4):
    xt = jnp.transpose(x, (0, 2, 3, 1)).astype(jnp.bfloat16)
    h1 = _conv_layer(xt, _wmat(w1), b1.reshape(1, -1), 96, 64)
    h2 = _conv_layer(h1, _wmat(w2), b2.reshape(1, -1), 64, 64)
    w4r = w4.reshape(2, 32).astype(jnp.bfloat16).astype(jnp.float32)
    dxa, dya = _head_layer(h2, _wmat(w3), b3.reshape(1, -1),
                           w4r[0:1], w4r[1:2], b4, 64, 32)
    direction_vectors = jnp.stack([dxa, dya], axis=1)
    centers = _centers(label, dxa, dya)
    return direction_vectors, centers


# speedup vs baseline: 4.6978x; 1.1124x over previous
import functools

import jax
import jax.numpy as jnp
from jax import lax
from jax.experimental import pallas as pl
from jax.experimental.pallas import tpu as pltpu

_H = 384
_W = 384
_TH = 32
_THRESHOLD = 100.0


def _build_cat(prev_ref, cur_ref, next_ref, slab_ref, cin):
    i = pl.program_id(1)
    nh = pl.num_programs(1)

    slab_ref[1:_TH + 1] = cur_ref[0]

    @pl.when(i == 0)
    def _():
        slab_ref[0:1] = jnp.zeros((1, _W, cin), slab_ref.dtype)

    @pl.when(i > 0)
    def _():
        slab_ref[0:1] = prev_ref[0, _TH - 1:_TH]

    @pl.when(i == nh - 1)
    def _():
        slab_ref[_TH + 1:_TH + 2] = jnp.zeros((1, _W, cin), slab_ref.dtype)

    @pl.when(i < nh - 1)
    def _():
        slab_ref[_TH + 1:_TH + 2] = next_ref[0, 0:1]

    return jnp.concatenate(
        [slab_ref[0:_TH], slab_ref[1:_TH + 1], slab_ref[2:_TH + 2]], axis=-1)


def _conv_matmul(cat, wm_ref, b_ref, cin, cout):
    y = jnp.dot(cat.reshape(_TH * _W, 3 * cin).astype(jnp.bfloat16),
                wm_ref[...],
                preferred_element_type=jnp.float32).reshape(_TH, _W, 3 * cout)
    y0 = y[:, :, 0:cout]
    y1 = y[:, :, cout:2 * cout]
    y2 = y[:, :, 2 * cout:3 * cout]
    col = lax.broadcasted_iota(jnp.int32, (_TH, _W, cout), 1)
    left = jnp.where(col >= 1, jnp.roll(y0, 1, axis=1), 0.0)
    right = jnp.where(col <= _W - 2, jnp.roll(y2, -1, axis=1), 0.0)
    return left + y1 + right + b_ref[0][None, None, :]


def _conv_body(prev_ref, cur_ref, next_ref, wm_ref, b_ref, out_ref, slab_ref,
               *, cin, cout):
    cat = _build_cat(prev_ref, cur_ref, next_ref, slab_ref, cin)
    s = _conv_matmul(cat, wm_ref, b_ref, cin, cout)
    out_ref[0] = jnp.maximum(s, 0.0).astype(jnp.bfloat16)


def _conv_layer(xpad, wm, bias, cin, cout):
    b = xpad.shape[0]
    nh = _H // _TH
    dt = xpad.dtype
    spec_in = lambda f: pl.BlockSpec((1, _TH, _W, cin), f)
    return pl.pallas_call(
        functools.partial(_conv_body, cin=cin, cout=cout),
        grid=(b, nh),
        in_specs=[
            spec_in(lambda bb, i: (bb, jnp.maximum(i - 1, 0), 0, 0)),
            spec_in(lambda bb, i: (bb, i, 0, 0)),
            spec_in(lambda bb, i: (bb, jnp.minimum(i + 1, nh - 1), 0, 0)),
            pl.BlockSpec((3 * cin, 3 * cout), lambda bb, i: (0, 0)),
            pl.BlockSpec((1, cout), lambda bb, i: (0, 0)),
        ],
        out_specs=pl.BlockSpec((1, _TH, _W, cout), lambda bb, i: (bb, i, 0, 0)),
        out_shape=jax.ShapeDtypeStruct((b, _H, _W, cout), jnp.bfloat16),
        scratch_shapes=[pltpu.VMEM((_TH + 2, _W, cin), dt)],
        compiler_params=pltpu.CompilerParams(
            dimension_semantics=("parallel", "parallel")),
    )(xpad, xpad, xpad, wm, bias)


def _head_body(prev_ref, cur_ref, next_ref, wm_ref, b_ref, w4x_ref, w4y_ref,
               b4_ref, dx_ref, dy_ref, slab_ref, *, cin, cout):
    cat = _build_cat(prev_ref, cur_ref, next_ref, slab_ref, cin)
    s = _conv_matmul(cat, wm_ref, b_ref, cin, cout)
    h3 = jnp.maximum(s, 0.0)
    h3b = h3.astype(jnp.bfloat16).astype(jnp.float32)
    d0 = jnp.sum(h3b * w4x_ref[0][None, None, :], axis=-1) + b4_ref[0]
    d1 = jnp.sum(h3b * w4y_ref[0][None, None, :], axis=-1) + b4_ref[1]
    inv = lax.rsqrt(jnp.maximum(d0 * d0 + d1 * d1, 1e-24))
    dx_ref[0] = d0 * inv
    dy_ref[0] = d1 * inv


def _head_layer(xpad, wm, bias, w4x, w4y, b4, cin, cout):
    b = xpad.shape[0]
    nh = _H // _TH
    spec_in = lambda f: pl.BlockSpec((1, _TH, _W, cin), f)
    out = jax.ShapeDtypeStruct((b, _H, _W), jnp.float32)
    return pl.pallas_call(
        functools.partial(_head_body, cin=cin, cout=cout),
        grid=(b, nh),
        in_specs=[
            spec_in(lambda bb, i: (bb, jnp.maximum(i - 1, 0), 0, 0)),
            spec_in(lambda bb, i: (bb, i, 0, 0)),
            spec_in(lambda bb, i: (bb, jnp.minimum(i + 1, nh - 1), 0, 0)),
            pl.BlockSpec((3 * cin, 3 * cout), lambda bb, i: (0, 0)),
            pl.BlockSpec((1, cout), lambda bb, i: (0, 0)),
            pl.BlockSpec((1, cout), lambda bb, i: (0, 0)),
            pl.BlockSpec((1, cout), lambda bb, i: (0, 0)),
            pl.BlockSpec(memory_space=pltpu.SMEM),
        ],
        out_specs=(
            pl.BlockSpec((1, _TH, _W), lambda bb, i: (bb, i, 0)),
            pl.BlockSpec((1, _TH, _W), lambda bb, i: (bb, i, 0)),
        ),
        out_shape=(out, out),
        scratch_shapes=[pltpu.VMEM((_TH + 2, _W, cin), jnp.bfloat16)],
        compiler_params=pltpu.CompilerParams(
            dimension_semantics=("parallel", "parallel")),
    )(xpad, xpad, xpad, wm, bias, w4x, w4y, b4)


def _roll2(t, j, k):
    if j:
        t = jnp.roll(t, j, axis=0)
    if k:
        t = jnp.roll(t, k, axis=1)
    return t


def _centers_body(label_ref, dx_ref, dy_ref, out_ref):
    lab = label_ref[0]
    dx = dx_ref[0]
    dy = dy_ref[0]
    xi = lax.broadcasted_iota(jnp.int32, (_H, _W), 1)
    yi = lax.broadcasted_iota(jnp.int32, (_H, _W), 0)
    xs = xi.astype(jnp.float32)
    ys = yi.astype(jnp.float32)
    vx = jnp.clip(xs + dx, 0.0, float(_W - 1)).astype(jnp.int32)
    vy = jnp.clip(ys + dy, 0.0, float(_H - 1)).astype(jnp.int32)
    rx = vx - xi
    ry = vy - yi
    flat = yi * _W + xi
    for oi, obj in enumerate((1, 2)):
        hough = jnp.zeros((_H, _W), jnp.float32)
        for j in (-1, 0, 1):
            for k in (-1, 0, 1):
                t = jnp.where((lab == obj) & (ry == j) & (rx == k), 1.0, 0.0)
                hough = hough + _roll2(t, j, k)
        mx = jnp.max(hough)
        idx = jnp.min(jnp.where(hough == mx, flat, _H * _W))
        found = mx > _THRESHOLD
        out_ref[0, oi, 0] = jnp.where(found, idx % _W, -1)
        out_ref[0, oi, 1] = jnp.where(found, idx // _W, -1)


def _centers(label, dxa, dya):
    b = label.shape[0]
    return pl.pallas_call(
        _centers_body,
        grid=(b,),
        in_specs=[
            pl.BlockSpec((1, _H, _W), lambda bb: (bb, 0, 0)),
            pl.BlockSpec((1, _H, _W), lambda bb: (bb, 0, 0)),
            pl.BlockSpec((1, _H, _W), lambda bb: (bb, 0, 0)),
        ],
        out_specs=pl.BlockSpec((1, 2, 2), lambda bb: (bb, 0, 0),
                               memory_space=pltpu.SMEM),
        out_shape=jax.ShapeDtypeStruct((b, 2, 2), jnp.int32),
        compiler_params=pltpu.CompilerParams(
            dimension_semantics=("parallel",)),
    )(label, dxa, dya)


def _wmat(w):
    return jnp.transpose(w, (2, 1, 3, 0)).reshape(
        3 * w.shape[1], 3 * w.shape[0]).astype(jnp.bfloat16)


def kernel(x, label, w1, b1, w2, b2, w3, b3, w4, b4):
    xt = jnp.transpose(x, (0, 2, 3, 1)).astype(jnp.bfloat16)
    h1 = _conv_layer(xt, _wmat(w1), b1.reshape(1, -1), 96, 64)
    h2 = _conv_layer(h1, _wmat(w2), b2.reshape(1, -1), 64, 64)
    w4r = w4.reshape(2, 32).astype(jnp.bfloat16).astype(jnp.float32)
    dxa, dya = _head_layer(h2, _wmat(w3), b3.reshape(1, -1),
                           w4r[0:1], w4r[1:2], b4, 64, 32)
    direction_vectors = jnp.stack([dxa, dya], axis=1)
    centers = _centers(label, dxa, dya)
    return direction_vectors, centers


# speedup vs baseline: 4.6981x; 1.0001x over previous
import functools

import jax
import jax.numpy as jnp
from jax import lax
from jax.experimental import pallas as pl
from jax.experimental.pallas import tpu as pltpu

_H = 384
_W = 384
_TH = 32
_THRESHOLD = 100.0


def _build_cat(prev_ref, cur_ref, next_ref, slab_ref, cin):
    i = pl.program_id(1)
    nh = pl.num_programs(1)

    slab_ref[1:_TH + 1] = cur_ref[0]

    @pl.when(i == 0)
    def _():
        slab_ref[0:1] = jnp.zeros((1, _W, cin), slab_ref.dtype)

    @pl.when(i > 0)
    def _():
        slab_ref[0:1] = prev_ref[0, _TH - 1:_TH]

    @pl.when(i == nh - 1)
    def _():
        slab_ref[_TH + 1:_TH + 2] = jnp.zeros((1, _W, cin), slab_ref.dtype)

    @pl.when(i < nh - 1)
    def _():
        slab_ref[_TH + 1:_TH + 2] = next_ref[0, 0:1]

    return jnp.concatenate(
        [slab_ref[0:_TH], slab_ref[1:_TH + 1], slab_ref[2:_TH + 2]], axis=-1)


def _conv_matmul(cat, wm_ref, b_ref, cin, cout):
    y = jnp.dot(cat.reshape(_TH * _W, 3 * cin).astype(jnp.bfloat16),
                wm_ref[...],
                preferred_element_type=jnp.float32).reshape(_TH, _W, 3 * cout)
    y0 = y[:, :, 0:cout]
    y1 = y[:, :, cout:2 * cout]
    y2 = y[:, :, 2 * cout:3 * cout]
    col = lax.broadcasted_iota(jnp.int32, (_TH, _W, cout), 1)
    left = jnp.where(col >= 1, jnp.roll(y0, 1, axis=1), 0.0)
    right = jnp.where(col <= _W - 2, jnp.roll(y2, -1, axis=1), 0.0)
    return left + y1 + right + b_ref[0][None, None, :]


def _conv_body(prev_ref, cur_ref, next_ref, wm_ref, b_ref, out_ref, slab_ref,
               *, cin, cout):
    cat = _build_cat(prev_ref, cur_ref, next_ref, slab_ref, cin)
    s = _conv_matmul(cat, wm_ref, b_ref, cin, cout)
    out_ref[0] = jnp.maximum(s, 0.0).astype(jnp.bfloat16)


def _conv_layer(xpad, wm, bias, cin, cout):
    b = xpad.shape[0]
    nh = _H // _TH
    dt = xpad.dtype
    spec_in = lambda f: pl.BlockSpec((1, _TH, _W, cin), f)
    return pl.pallas_call(
        functools.partial(_conv_body, cin=cin, cout=cout),
        grid=(b, nh),
        in_specs=[
            spec_in(lambda bb, i: (bb, jnp.maximum(i - 1, 0), 0, 0)),
            spec_in(lambda bb, i: (bb, i, 0, 0)),
            spec_in(lambda bb, i: (bb, jnp.minimum(i + 1, nh - 1), 0, 0)),
            pl.BlockSpec((3 * cin, 3 * cout), lambda bb, i: (0, 0)),
            pl.BlockSpec((1, cout), lambda bb, i: (0, 0)),
        ],
        out_specs=pl.BlockSpec((1, _TH, _W, cout), lambda bb, i: (bb, i, 0, 0)),
        out_shape=jax.ShapeDtypeStruct((b, _H, _W, cout), jnp.bfloat16),
        scratch_shapes=[pltpu.VMEM((_TH + 2, _W, cin), dt)],
        compiler_params=pltpu.CompilerParams(
            dimension_semantics=("parallel", "parallel")),
    )(xpad, xpad, xpad, wm, bias)


def _head_body(prev_ref, cur_ref, next_ref, wm_ref, b_ref, w4x_ref, w4y_ref,
               b4_ref, dx_ref, dy_ref, slab_ref, *, cin, cout):
    cat = _build_cat(prev_ref, cur_ref, next_ref, slab_ref, cin)
    s = _conv_matmul(cat, wm_ref, b_ref, cin, cout)
    h3 = jnp.maximum(s, 0.0)
    h3b = h3.astype(jnp.bfloat16).astype(jnp.float32)
    d0 = jnp.sum(h3b * w4x_ref[0][None, None, :], axis=-1) + b4_ref[0]
    d1 = jnp.sum(h3b * w4y_ref[0][None, None, :], axis=-1) + b4_ref[1]
    inv = lax.rsqrt(jnp.maximum(d0 * d0 + d1 * d1, 1e-24))
    dx_ref[0] = d0 * inv
    dy_ref[0] = d1 * inv


def _head_layer(xpad, wm, bias, w4x, w4y, b4, cin, cout):
    b = xpad.shape[0]
    nh = _H // _TH
    spec_in = lambda f: pl.BlockSpec((1, _TH, _W, cin), f)
    out = jax.ShapeDtypeStruct((b, _H, _W), jnp.float32)
    return pl.pallas_call(
        functools.partial(_head_body, cin=cin, cout=cout),
        grid=(b, nh),
        in_specs=[
            spec_in(lambda bb, i: (bb, jnp.maximum(i - 1, 0), 0, 0)),
            spec_in(lambda bb, i: (bb, i, 0, 0)),
            spec_in(lambda bb, i: (bb, jnp.minimum(i + 1, nh - 1), 0, 0)),
            pl.BlockSpec((3 * cin, 3 * cout), lambda bb, i: (0, 0)),
            pl.BlockSpec((1, cout), lambda bb, i: (0, 0)),
            pl.BlockSpec((1, cout), lambda bb, i: (0, 0)),
            pl.BlockSpec((1, cout), lambda bb, i: (0, 0)),
            pl.BlockSpec(memory_space=pltpu.SMEM),
        ],
        out_specs=(
            pl.BlockSpec((1, _TH, _W), lambda bb, i: (bb, i, 0)),
            pl.BlockSpec((1, _TH, _W), lambda bb, i: (bb, i, 0)),
        ),
        out_shape=(out, out),
        scratch_shapes=[pltpu.VMEM((_TH + 2, _W, cin), jnp.bfloat16)],
        compiler_params=pltpu.CompilerParams(
            dimension_semantics=("parallel", "parallel")),
    )(xpad, xpad, xpad, wm, bias, w4x, w4y, b4)


def _roll2(t, j, k):
    if j:
        t = jnp.roll(t, j, axis=0)
    if k:
        t = jnp.roll(t, k, axis=1)
    return t


def _centers_body(label_ref, dx_ref, dy_ref, out_ref):
    lab = label_ref[0]
    dx = dx_ref[0]
    dy = dy_ref[0]
    xi = lax.broadcasted_iota(jnp.int32, (_H, _W), 1)
    yi = lax.broadcasted_iota(jnp.int32, (_H, _W), 0)
    xs = xi.astype(jnp.float32)
    ys = yi.astype(jnp.float32)
    vx = jnp.clip(xs + dx, 0.0, float(_W - 1)).astype(jnp.int32)
    vy = jnp.clip(ys + dy, 0.0, float(_H - 1)).astype(jnp.int32)
    rx = vx - xi
    ry = vy - yi
    flat = yi * _W + xi
    for oi, obj in enumerate((1, 2)):
        hough = jnp.zeros((_H, _W), jnp.float32)
        for j in (-1, 0, 1):
            for k in (-1, 0, 1):
                t = jnp.where((lab == obj) & (ry == j) & (rx == k), 1.0, 0.0)
                hough = hough + _roll2(t, j, k)
        mx = jnp.max(hough)
        idx = jnp.min(jnp.where(hough == mx, flat, _H * _W))
        found = mx > _THRESHOLD
        out_ref[0, oi, 0] = jnp.where(found, idx % _W, -1)
        out_ref[0, oi, 1] = jnp.where(found, idx // _W, -1)


def _centers(label, dxa, dya):
    b = label.shape[0]
    return pl.pallas_call(
        _centers_body,
        grid=(b,),
        in_specs=[
            pl.BlockSpec((1, _H, _W), lambda bb: (bb, 0, 0)),
            pl.BlockSpec((1, _H, _W), lambda bb: (bb, 0, 0)),
            pl.BlockSpec((1, _H, _W), lambda bb: (bb, 0, 0)),
        ],
        out_specs=pl.BlockSpec((1, 2, 2), lambda bb: (bb, 0, 0),
                               memory_space=pltpu.SMEM),
        out_shape=jax.ShapeDtypeStruct((b, 2, 2), jnp.int32),
        compiler_params=pltpu.CompilerParams(
            dimension_semantics=("parallel",)),
    )(label, dxa, dya)


def _wmat(w):
    return jnp.transpose(w, (2, 1, 3, 0)).reshape(
        3 * w.shape[1], 3 * w.shape[0]).astype(jnp.bfloat16)


def kernel(x, label, w1, b1, w2, b2, w3, b3, w4, b4):
    xt = jnp.transpose(x.astype(jnp.bfloat16), (0, 2, 3, 1))
    h1 = _conv_layer(xt, _wmat(w1), b1.reshape(1, -1), 96, 64)
    h2 = _conv_layer(h1, _wmat(w2), b2.reshape(1, -1), 64, 64)
    w4r = w4.reshape(2, 32).astype(jnp.bfloat16).astype(jnp.float32)
    dxa, dya = _head_layer(h2, _wmat(w3), b3.reshape(1, -1),
                           w4r[0:1], w4r[1:2], b4, 64, 32)
    direction_vectors = jnp.stack([dxa, dya], axis=1)
    centers = _centers(label, dxa, dya)
    return direction_vectors, centers
